# baseline (speedup 1.0000x reference)
import functools

import jax
import jax.numpy as jnp
from jax.experimental import pallas as pl
from jax.experimental.pallas import tpu as pltpu


_ROWS = 2048
_COLS = 8192
_CH = 64
_NBUF = 8


def _make_body(row0, nrows):
    num = nrows // _CH
    ngrp = num // _NBUF

    def body(w_hbm, t_hbm, o_hbm, w_buf, o_buf, t_v, in_sems, out_sems, t_sem):
        def in_copy(i, b):
            return pltpu.make_async_copy(
                w_hbm.at[pl.ds(row0 + i * _CH, _CH), :], w_buf.at[b],
                in_sems.at[b])

        def out_copy(i, b):
            return pltpu.make_async_copy(
                o_buf.at[b], o_hbm.at[pl.ds(i * _CH, _CH), :], out_sems.at[b])

        for b in range(_NBUF):
            in_copy(b, b).start()

        cp = pltpu.make_async_copy(t_hbm.at[pl.ds(row0, nrows), :], t_v, t_sem)
        cp.start()
        cp.wait()
        t_v[...] = jax.nn.sigmoid(t_v[...])

        def grp(g, carry):
            for b in range(_NBUF):
                i = g * _NBUF + b
                in_copy(i, b).wait()

                @pl.when(g > 0)
                def _():
                    out_copy(i - _NBUF, b).wait()

                w = w_buf[b]
                s = t_v[pl.ds(i * _CH, _CH), :]
                o_buf[b] = jnp.maximum(w - s, 0.0) + jnp.minimum(w + s, 0.0)
                out_copy(i, b).start()

                @pl.when(g < ngrp - 1)
                def _():
                    in_copy(i + _NBUF, b).start()

            return carry

        jax.lax.fori_loop(0, ngrp, grp, 0)

        for b in range(_NBUF):
            out_copy((ngrp - 1) * _NBUF + b, b).wait()

    return body


def _masked_rows(weight, threshold, row0, nrows):
    return pl.pallas_call(
        _make_body(row0, nrows),
        in_specs=[
            pl.BlockSpec(memory_space=pl.ANY),
            pl.BlockSpec(memory_space=pl.ANY),
        ],
        out_specs=pl.BlockSpec(memory_space=pl.ANY),
        out_shape=jax.ShapeDtypeStruct((nrows, _COLS), weight.dtype),
        scratch_shapes=[
            pltpu.VMEM((_NBUF, _CH, _COLS), jnp.float32),
            pltpu.VMEM((_NBUF, _CH, _COLS), jnp.float32),
            pltpu.VMEM((nrows, 1), jnp.float32),
            pltpu.SemaphoreType.DMA((_NBUF,)),
            pltpu.SemaphoreType.DMA((_NBUF,)),
            pltpu.SemaphoreType.DMA,
        ],
    )(weight, threshold)


def kernel(weight, threshold, alpha):
    half = _ROWS // 2
    out0 = _masked_rows(weight, threshold, 0, half)
    out1 = _masked_rows(weight, threshold, half, _ROWS - half)
    return jnp.concatenate([out0, out1], axis=0)


# speedup vs baseline: 1.9612x; 1.9612x over previous
import functools

import jax
import jax.numpy as jnp
from jax.experimental import pallas as pl
from jax.experimental.pallas import tpu as pltpu


_ROWS = 2048
_COLS = 8192
_CH = 64
_NBUF = 8


def _make_body(row0, nrows):
    num = nrows // _CH
    ngrp = num // _NBUF

    def body(w_hbm, t_hbm, o_hbm, w_buf, o_buf, t_v, in_sems, out_sems, t_sem):
        def in_copy(i, b):
            return pltpu.make_async_copy(
                w_hbm.at[pl.ds(row0 + i * _CH, _CH), :], w_buf.at[b],
                in_sems.at[b])

        def out_copy(i, b):
            return pltpu.make_async_copy(
                o_buf.at[b], o_hbm.at[pl.ds(i * _CH, _CH), :], out_sems.at[b])

        for b in range(_NBUF):
            in_copy(b, b).start()

        cp = pltpu.make_async_copy(t_hbm.at[pl.ds(row0, nrows), :], t_v, t_sem)
        cp.start()
        cp.wait()
        t_v[...] = jax.nn.sigmoid(t_v[...])

        for i in range(num):
            b = i % _NBUF
            in_copy(i, b).wait()
            if i >= _NBUF:
                out_copy(i - _NBUF, b).wait()
            w = w_buf[b]
            s = t_v[pl.ds(i * _CH, _CH), :]
            o_buf[b] = jnp.maximum(w - s, 0.0) + jnp.minimum(w + s, 0.0)
            out_copy(i, b).start()
            if i + _NBUF < num:
                in_copy(i + _NBUF, b).start()

        for i in range(num - _NBUF, num):
            out_copy(i, i % _NBUF).wait()

    return body


def _masked_rows(weight, threshold, row0, nrows):
    return pl.pallas_call(
        _make_body(row0, nrows),
        in_specs=[
            pl.BlockSpec(memory_space=pl.ANY),
            pl.BlockSpec(memory_space=pl.ANY),
        ],
        out_specs=pl.BlockSpec(memory_space=pl.ANY),
        out_shape=jax.ShapeDtypeStruct((nrows, _COLS), weight.dtype),
        scratch_shapes=[
            pltpu.VMEM((_NBUF, _CH, _COLS), jnp.float32),
            pltpu.VMEM((_NBUF, _CH, _COLS), jnp.float32),
            pltpu.VMEM((nrows, 1), jnp.float32),
            pltpu.SemaphoreType.DMA((_NBUF,)),
            pltpu.SemaphoreType.DMA((_NBUF,)),
            pltpu.SemaphoreType.DMA,
        ],
    )(weight, threshold)


def kernel(weight, threshold, alpha):
    return _masked_rows(weight, threshold, 0, _ROWS)
